# baseline (speedup 1.0000x reference)
import functools
import jax
import jax.numpy as jnp
from jax import lax
from jax.experimental import pallas as pl
from jax.experimental.pallas import tpu as pltpu
from jax.experimental.pallas import tpu_sc as plsc

BM = 512
BMC = 2048
CP = 16


def _knn_idx_kernel(pT_ref, q_ref, gidx_ref, wspl_ref, *, n_src, jb):
    i = pl.program_id(0)
    b = i // jb

    q_blk = q_ref[0]
    pT = pT_ref[0]

    d0 = q_blk[:, 0:1] - pT[0:1, :]
    d1 = q_blk[:, 1:2] - pT[1:2, :]
    d2 = q_blk[:, 2:3] - pT[2:3, :]
    d = d0 * d0 + d1 * d1 + d2 * d2

    inf = jnp.float32(jnp.inf)
    big = jnp.int32(n_src)
    iota = lax.broadcasted_iota(jnp.int32, (BM, n_src), 1)

    t1 = jnp.min(d, axis=1, keepdims=True)
    e1 = d == t1
    dm1 = jnp.where(e1, inf, d)
    t2 = jnp.min(dm1, axis=1, keepdims=True)
    e2 = dm1 == t2
    dm2 = jnp.where(e2, inf, dm1)
    t3 = jnp.min(dm2, axis=1, keepdims=True)
    e3 = dm1 == t3

    i1 = jnp.min(jnp.where(e1, iota, big), axis=1, keepdims=True)
    i2 = jnp.min(jnp.where(e2, iota, big), axis=1, keepdims=True)
    i3 = jnp.min(jnp.where(e3, iota, big), axis=1, keepdims=True)

    w1 = 1.0 / jnp.maximum(t1, 1e-10)
    w2 = 1.0 / jnp.maximum(t2, 1e-10)
    w3 = 1.0 / jnp.maximum(t3, 1e-10)
    wsum = w1 + w2 + w3
    wn1 = w1 / wsum
    wn2 = w2 / wsum
    wn3 = w3 / wsum

    off = jnp.int32(n_src) * b
    gidx_ref[0] = jnp.concatenate([i1, i2, i3], axis=1) + off
    ones = jnp.ones((1, 16), jnp.float32)
    wspl_ref[0] = jnp.concatenate(
        [wn1 * ones, wn2 * ones, wn3 * ones], axis=1)


def _sc_interp_kernel(xT_hbm, gidx_hbm, wspl_hbm, xi_hbm,
                      idx_v, rows_v, w_v, out_v, sem, *, npts_w, cx):
    wid = lax.axis_index("s") * 2 + lax.axis_index("c")
    base = wid * npts_w
    nchunks = npts_w // CP

    def chunk(c, carry):
        base_pt = base + c * CP
        pltpu.sync_copy(gidx_hbm.at[pl.ds(base_pt * 3, 3 * CP)], idx_v)
        pltpu.async_copy(xT_hbm.at[idx_v], rows_v, sem).wait()
        pltpu.sync_copy(wspl_hbm.at[pl.ds(base_pt * 3, 3 * CP)], w_v)

        def point(t, carry2):
            for v in range(cx // 16):
                sl = pl.ds(v * 16, 16)
                acc = (rows_v[3 * t, sl] * w_v[3 * t]
                       + rows_v[3 * t + 1, sl] * w_v[3 * t + 1]
                       + rows_v[3 * t + 2, sl] * w_v[3 * t + 2])
                out_v[t, sl] = acc
            return carry2

        lax.fori_loop(0, CP, point, 0)
        pltpu.sync_copy(out_v, xi_hbm.at[pl.ds(base_pt, CP)])
        return carry

    lax.fori_loop(0, nchunks, chunk, 0)


def _l1_kernel(xi_ref, y_ref, Wx_ref, Wy_ref, b1_ref,
               h1_ref, s1_ref, ss1_ref):
    b = pl.program_id(0)
    j = pl.program_id(1)
    h1 = (lax.dot_general(Wx_ref[...], xi_ref[0].astype(jnp.bfloat16),
                          dimension_numbers=(((1,), (1,)), ((), ())),
                          preferred_element_type=jnp.float32)
          + lax.dot_general(Wy_ref[...], y_ref[0].astype(jnp.bfloat16),
                            dimension_numbers=(((1,), (0,)), ((), ())),
                            preferred_element_type=jnp.float32)
          + b1_ref[...])
    h1_ref[0] = h1.astype(jnp.bfloat16)

    @pl.when(jnp.logical_and(b == 0, j == 0))
    def _():
        s1_ref[...] = jnp.zeros_like(s1_ref)
        ss1_ref[...] = jnp.zeros_like(ss1_ref)

    s1_ref[...] += jnp.sum(h1, axis=1, keepdims=True)
    ss1_ref[...] += jnp.sum(h1 * h1, axis=1, keepdims=True)


def _bn_l2_kernel(h1_ref, s1_ref, ss1_ref, g1_ref, be1_ref, W2_ref, b2_ref,
                  h2_ref, s2_ref, ss2_ref, *, count):
    b = pl.program_id(0)
    j = pl.program_id(1)

    mean = s1_ref[...] / count
    var = ss1_ref[...] / count - mean * mean
    rstd = lax.rsqrt(var + 1e-5)
    scale = g1_ref[...] * rstd
    shift = be1_ref[...] - mean * scale

    h1f = h1_ref[0].astype(jnp.float32)
    h1 = jnp.maximum(h1f * scale + shift, 0.0)
    h2 = (lax.dot_general(W2_ref[...], h1.astype(jnp.bfloat16),
                          dimension_numbers=(((1,), (0,)), ((), ())),
                          preferred_element_type=jnp.float32)
          + b2_ref[...])
    h2_ref[0] = h2.astype(jnp.bfloat16)

    @pl.when(jnp.logical_and(b == 0, j == 0))
    def _():
        s2_ref[...] = jnp.zeros_like(s2_ref)
        ss2_ref[...] = jnp.zeros_like(ss2_ref)

    s2_ref[...] += jnp.sum(h2, axis=1, keepdims=True)
    ss2_ref[...] += jnp.sum(h2 * h2, axis=1, keepdims=True)


def _bn_out_kernel(h2_ref, s2_ref, ss2_ref, g2_ref, be2_ref, out_ref, *,
                   count):
    mean = s2_ref[...] / count
    var = ss2_ref[...] / count - mean * mean
    rstd = lax.rsqrt(var + 1e-5)
    scale = g2_ref[...] * rstd
    shift = be2_ref[...] - mean * scale
    h2f = h2_ref[0].astype(jnp.float32)
    out_ref[0] = jnp.maximum(h2f * scale + shift, 0.0)


def kernel(p, q, x, y, W1, b1, g1, be1, W2, b2, g2, be2):
    B, N, _ = p.shape
    M = q.shape[1]
    Cx = x.shape[1]
    Cy = y.shape[1]
    C1 = W1.shape[0]
    C2 = W2.shape[0]
    count = float(B * M)
    jb = M // BM
    grid_a = (B * jb,)

    pT = jnp.swapaxes(p, 1, 2)
    xT = jnp.reshape(jnp.transpose(x, (0, 2, 1)), (B * N, Cx))
    Wx = W1[:, :Cx].astype(jnp.bfloat16)
    Wy = W1[:, Cx:].astype(jnp.bfloat16)
    W2_bf = W2.astype(jnp.bfloat16)
    col = lambda v: v.reshape(-1, 1)

    gidx, wspl = pl.pallas_call(
        functools.partial(_knn_idx_kernel, n_src=N, jb=jb),
        grid=grid_a,
        in_specs=[
            pl.BlockSpec((1, 3, N), lambda i: (i // (M // BM), 0, 0)),
            pl.BlockSpec((1, BM, 3), lambda i: (i // (M // BM),
                                                i % (M // BM), 0)),
        ],
        out_specs=[
            pl.BlockSpec((1, BM, 3), lambda i: (i // (M // BM),
                                                i % (M // BM), 0)),
            pl.BlockSpec((1, BM, 48), lambda i: (i // (M // BM),
                                                 i % (M // BM), 0)),
        ],
        out_shape=[
            jax.ShapeDtypeStruct((B, M, 3), jnp.int32),
            jax.ShapeDtypeStruct((B, M, 48), jnp.float32),
        ],
    )(pT, q)

    NW = 32
    npts_w = (B * M) // NW
    mesh = plsc.VectorSubcoreMesh(core_axis_name="c", subcore_axis_name="s")
    xi = pl.kernel(
        functools.partial(_sc_interp_kernel, npts_w=npts_w, cx=Cx),
        mesh=mesh,
        out_type=jax.ShapeDtypeStruct((B * M, Cx), jnp.float32),
        scratch_types=[
            pltpu.VMEM((3 * CP,), jnp.int32),
            pltpu.VMEM((3 * CP, Cx), jnp.float32),
            pltpu.VMEM((3 * CP, 16), jnp.float32),
            pltpu.VMEM((CP, Cx), jnp.float32),
            pltpu.SemaphoreType.DMA,
        ],
    )(xT, jnp.reshape(gidx, (B * M * 3,)),
      jnp.reshape(wspl, (B * M * 3, 16)))
    xi = jnp.reshape(xi, (B, M, Cx))

    grid_l1 = (B, M // BM)
    h1_pre, s1, ss1 = pl.pallas_call(
        _l1_kernel,
        grid=grid_l1,
        in_specs=[
            pl.BlockSpec((1, BM, Cx), lambda b, j: (b, j, 0)),
            pl.BlockSpec((1, Cy, BM), lambda b, j: (b, 0, j)),
            pl.BlockSpec((C1, Cx), lambda b, j: (0, 0)),
            pl.BlockSpec((C1, Cy), lambda b, j: (0, 0)),
            pl.BlockSpec((C1, 1), lambda b, j: (0, 0)),
        ],
        out_specs=[
            pl.BlockSpec((1, C1, BM), lambda b, j: (b, 0, j)),
            pl.BlockSpec((C1, 1), lambda b, j: (0, 0)),
            pl.BlockSpec((C1, 1), lambda b, j: (0, 0)),
        ],
        out_shape=[
            jax.ShapeDtypeStruct((B, C1, M), jnp.bfloat16),
            jax.ShapeDtypeStruct((C1, 1), jnp.float32),
            jax.ShapeDtypeStruct((C1, 1), jnp.float32),
        ],
    )(xi, y, Wx, Wy, col(b1))

    grid_bc = (B, M // BMC)
    h2_pre, s2, ss2 = pl.pallas_call(
        functools.partial(_bn_l2_kernel, count=count),
        grid=grid_bc,
        in_specs=[
            pl.BlockSpec((1, C1, BMC), lambda b, j: (b, 0, j)),
            pl.BlockSpec((C1, 1), lambda b, j: (0, 0)),
            pl.BlockSpec((C1, 1), lambda b, j: (0, 0)),
            pl.BlockSpec((C1, 1), lambda b, j: (0, 0)),
            pl.BlockSpec((C1, 1), lambda b, j: (0, 0)),
            pl.BlockSpec((C2, C1), lambda b, j: (0, 0)),
            pl.BlockSpec((C2, 1), lambda b, j: (0, 0)),
        ],
        out_specs=[
            pl.BlockSpec((1, C2, BMC), lambda b, j: (b, 0, j)),
            pl.BlockSpec((C2, 1), lambda b, j: (0, 0)),
            pl.BlockSpec((C2, 1), lambda b, j: (0, 0)),
        ],
        out_shape=[
            jax.ShapeDtypeStruct((B, C2, M), jnp.bfloat16),
            jax.ShapeDtypeStruct((C2, 1), jnp.float32),
            jax.ShapeDtypeStruct((C2, 1), jnp.float32),
        ],
    )(h1_pre, s1, ss1, col(g1), col(be1), W2_bf, col(b2))

    h = pl.pallas_call(
        functools.partial(_bn_out_kernel, count=count),
        grid=grid_bc,
        in_specs=[
            pl.BlockSpec((1, C2, BMC), lambda b, j: (b, 0, j)),
            pl.BlockSpec((C2, 1), lambda b, j: (0, 0)),
            pl.BlockSpec((C2, 1), lambda b, j: (0, 0)),
            pl.BlockSpec((C2, 1), lambda b, j: (0, 0)),
            pl.BlockSpec((C2, 1), lambda b, j: (0, 0)),
        ],
        out_specs=pl.BlockSpec((1, C2, BMC), lambda b, j: (b, 0, j)),
        out_shape=jax.ShapeDtypeStruct((B, C2, M), jnp.float32),
    )(h2_pre, s2, ss2, col(g2), col(be2))

    return (q, h)


# speedup vs baseline: 3.3803x; 3.3803x over previous
import functools
import jax
import jax.numpy as jnp
from jax.experimental import pallas as pl
from jax.experimental.pallas import tpu as pltpu

K = 3
BM = 1024
BMC = 2048


def _knn_l1_kernel(pT_ref, q_ref, x_ref, y_ref, Wx_ref, Wy_ref, b1_ref,
                   h1_ref, s1_ref, ss1_ref, st_ref, *, n_src, n_steps):
    i = pl.program_id(0)

    @pl.when(i == 0)
    def _():
        s1_ref[...] = jnp.zeros_like(s1_ref)
        ss1_ref[...] = jnp.zeros_like(ss1_ref)
        st_ref[1] = jnp.zeros_like(st_ref[1])

    ST_prev = st_ref[(i + 1) % 2]
    xi = jax.lax.dot_general(x_ref[0].astype(jnp.bfloat16), ST_prev,
                             dimension_numbers=(((1,), (1,)), ((), ())),
                             preferred_element_type=jnp.float32)
    h1 = (jax.lax.dot_general(Wx_ref[...], xi.astype(jnp.bfloat16),
                              dimension_numbers=(((1,), (0,)), ((), ())),
                              preferred_element_type=jnp.float32)
          + jax.lax.dot_general(Wy_ref[...], y_ref[0].astype(jnp.bfloat16),
                                dimension_numbers=(((1,), (0,)), ((), ())),
                                preferred_element_type=jnp.float32)
          + b1_ref[...])
    h1_ref[0] = h1.astype(jnp.bfloat16)

    live = jnp.where(i > 0, 1.0, 0.0).astype(jnp.float32)
    s1_ref[...] += live * jnp.sum(h1, axis=1, keepdims=True)
    ss1_ref[...] += live * jnp.sum(h1 * h1, axis=1, keepdims=True)

    q_blk = q_ref[0]
    pT = pT_ref[0]

    d0 = q_blk[:, 0:1] - pT[0:1, :]
    d1 = q_blk[:, 1:2] - pT[1:2, :]
    d2 = q_blk[:, 2:3] - pT[2:3, :]
    d = d0 * d0 + d1 * d1 + d2 * d2

    inf = jnp.float32(jnp.inf)
    t1 = jnp.min(d, axis=1, keepdims=True)
    dm1 = jnp.where(d == t1, inf, d)
    t2 = jnp.min(dm1, axis=1, keepdims=True)
    dm2 = jnp.where(dm1 == t2, inf, dm1)
    t3 = jnp.min(dm2, axis=1, keepdims=True)

    w1 = 1.0 / jnp.maximum(t1, 1e-10)
    w2 = 1.0 / jnp.maximum(t2, 1e-10)
    w3 = 1.0 / jnp.maximum(t3, 1e-10)
    wsum = w1 + w2 + w3
    wn1 = w1 / wsum
    wn2 = w2 / wsum
    wn3 = w3 / wsum

    ST = jnp.where(d == t1, wn1,
                   jnp.where(dm1 == t2, wn2,
                             jnp.where(dm1 == t3, wn3, 0.0))
                   ).astype(jnp.bfloat16)
    st_ref[i % 2] = ST


def _bn_l2_kernel(h1_ref, s1_ref, ss1_ref, g1_ref, be1_ref, W2_ref, b2_ref,
                  h2_ref, s2_ref, ss2_ref, *, count):
    b = pl.program_id(0)
    j = pl.program_id(1)

    mean = s1_ref[...] / count
    var = ss1_ref[...] / count - mean * mean
    rstd = jax.lax.rsqrt(var + 1e-5)
    scale = g1_ref[...] * rstd
    shift = be1_ref[...] - mean * scale

    h1f = h1_ref[0].astype(jnp.float32)
    h1 = jnp.maximum(h1f * scale + shift, 0.0)
    h2 = (jax.lax.dot_general(W2_ref[...], h1.astype(jnp.bfloat16),
                              dimension_numbers=(((1,), (0,)), ((), ())),
                              preferred_element_type=jnp.float32)
          + b2_ref[...])
    h2_ref[0] = h2.astype(jnp.bfloat16)

    @pl.when(jnp.logical_and(b == 0, j == 0))
    def _():
        s2_ref[...] = jnp.zeros_like(s2_ref)
        ss2_ref[...] = jnp.zeros_like(ss2_ref)

    s2_ref[...] += jnp.sum(h2, axis=1, keepdims=True)
    ss2_ref[...] += jnp.sum(h2 * h2, axis=1, keepdims=True)


def _bn_out_kernel(h2_ref, s2_ref, ss2_ref, g2_ref, be2_ref, out_ref, *,
                   count):
    mean = s2_ref[...] / count
    var = ss2_ref[...] / count - mean * mean
    rstd = jax.lax.rsqrt(var + 1e-5)
    scale = g2_ref[...] * rstd
    shift = be2_ref[...] - mean * scale
    h2f = h2_ref[0].astype(jnp.float32)
    out_ref[0] = jnp.maximum(h2f * scale + shift, 0.0)


def kernel(p, q, x, y, W1, b1, g1, be1, W2, b2, g2, be2):
    B, N, _ = p.shape
    M = q.shape[1]
    Cx = x.shape[1]
    Cy = y.shape[1]
    C1 = W1.shape[0]
    C2 = W2.shape[0]
    count = float(B * M)
    n_blocks = B * (M // BM)
    n_steps = n_blocks + 1
    jb = M // BM

    pT = jnp.swapaxes(p, 1, 2)
    Wx = W1[:, :Cx].astype(jnp.bfloat16)
    Wy = W1[:, Cx:].astype(jnp.bfloat16)
    W2_bf = W2.astype(jnp.bfloat16)
    col = lambda v: v.reshape(-1, 1)

    def cur(i):
        return jnp.minimum(i, n_blocks - 1)

    def prev(i):
        return jnp.maximum(i - 1, 0)

    h1_pre, s1, ss1 = pl.pallas_call(
        functools.partial(_knn_l1_kernel, n_src=N, n_steps=n_steps),
        grid=(n_steps,),
        in_specs=[
            pl.BlockSpec((1, 3, N), lambda i: (cur(i) // jb, 0, 0)),
            pl.BlockSpec((1, BM, 3), lambda i: (cur(i) // jb, cur(i) % jb, 0)),
            pl.BlockSpec((1, Cx, N), lambda i: (prev(i) // jb, 0, 0)),
            pl.BlockSpec((1, Cy, BM), lambda i: (prev(i) // jb, 0,
                                                 prev(i) % jb)),
            pl.BlockSpec((C1, Cx), lambda i: (0, 0)),
            pl.BlockSpec((C1, Cy), lambda i: (0, 0)),
            pl.BlockSpec((C1, 1), lambda i: (0, 0)),
        ],
        out_specs=[
            pl.BlockSpec((1, C1, BM), lambda i: (prev(i) // jb, 0,
                                                 prev(i) % jb)),
            pl.BlockSpec((C1, 1), lambda i: (0, 0)),
            pl.BlockSpec((C1, 1), lambda i: (0, 0)),
        ],
        out_shape=[
            jax.ShapeDtypeStruct((B, C1, M), jnp.bfloat16),
            jax.ShapeDtypeStruct((C1, 1), jnp.float32),
            jax.ShapeDtypeStruct((C1, 1), jnp.float32),
        ],
        scratch_shapes=[pltpu.VMEM((2, BM, N), jnp.bfloat16)],
    )(pT, q, x, y, Wx, Wy, col(b1))

    grid_bc = (B, M // BMC)
    h2_pre, s2, ss2 = pl.pallas_call(
        functools.partial(_bn_l2_kernel, count=count),
        grid=grid_bc,
        in_specs=[
            pl.BlockSpec((1, C1, BMC), lambda b, j: (b, 0, j)),
            pl.BlockSpec((C1, 1), lambda b, j: (0, 0)),
            pl.BlockSpec((C1, 1), lambda b, j: (0, 0)),
            pl.BlockSpec((C1, 1), lambda b, j: (0, 0)),
            pl.BlockSpec((C1, 1), lambda b, j: (0, 0)),
            pl.BlockSpec((C2, C1), lambda b, j: (0, 0)),
            pl.BlockSpec((C2, 1), lambda b, j: (0, 0)),
        ],
        out_specs=[
            pl.BlockSpec((1, C2, BMC), lambda b, j: (b, 0, j)),
            pl.BlockSpec((C2, 1), lambda b, j: (0, 0)),
            pl.BlockSpec((C2, 1), lambda b, j: (0, 0)),
        ],
        out_shape=[
            jax.ShapeDtypeStruct((B, C2, M), jnp.bfloat16),
            jax.ShapeDtypeStruct((C2, 1), jnp.float32),
            jax.ShapeDtypeStruct((C2, 1), jnp.float32),
        ],
    )(h1_pre, s1, ss1, col(g1), col(be1), W2_bf, col(b2))

    h = pl.pallas_call(
        functools.partial(_bn_out_kernel, count=count),
        grid=grid_bc,
        in_specs=[
            pl.BlockSpec((1, C2, BMC), lambda b, j: (b, 0, j)),
            pl.BlockSpec((C2, 1), lambda b, j: (0, 0)),
            pl.BlockSpec((C2, 1), lambda b, j: (0, 0)),
            pl.BlockSpec((C2, 1), lambda b, j: (0, 0)),
            pl.BlockSpec((C2, 1), lambda b, j: (0, 0)),
        ],
        out_specs=pl.BlockSpec((1, C2, BMC), lambda b, j: (b, 0, j)),
        out_shape=jax.ShapeDtypeStruct((B, C2, M), jnp.float32),
    )(h2_pre, s2, ss2, col(g2), col(be2))

    return (q, h)


# speedup vs baseline: 3.5482x; 1.0497x over previous
import functools
import jax
import jax.numpy as jnp
from jax.experimental import pallas as pl
from jax.experimental.pallas import tpu as pltpu

K = 3
BM = 1024
BMC = 4096


def _knn_l1_kernel(pT_ref, q_ref, x_ref, y_ref, Wx_ref, Wy_ref, b1_ref,
                   h1_ref, s1_ref, ss1_ref, st_ref, *, n_src, n_steps):
    i = pl.program_id(0)

    @pl.when(i == 0)
    def _():
        s1_ref[...] = jnp.zeros_like(s1_ref)
        ss1_ref[...] = jnp.zeros_like(ss1_ref)
        st_ref[1] = jnp.zeros_like(st_ref[1])

    ST_prev = st_ref[(i + 1) % 2]
    xi = jax.lax.dot_general(x_ref[0].astype(jnp.bfloat16), ST_prev,
                             dimension_numbers=(((1,), (1,)), ((), ())),
                             preferred_element_type=jnp.float32)
    h1 = (jax.lax.dot_general(Wx_ref[...], xi.astype(jnp.bfloat16),
                              dimension_numbers=(((1,), (0,)), ((), ())),
                              preferred_element_type=jnp.float32)
          + jax.lax.dot_general(Wy_ref[...], y_ref[0].astype(jnp.bfloat16),
                                dimension_numbers=(((1,), (0,)), ((), ())),
                                preferred_element_type=jnp.float32)
          + b1_ref[...])
    h1_ref[0] = h1.astype(jnp.bfloat16)

    live = jnp.where(i > 0, 1.0, 0.0).astype(jnp.float32)
    s1_ref[...] += live * jnp.sum(h1, axis=1, keepdims=True)
    ss1_ref[...] += live * jnp.sum(h1 * h1, axis=1, keepdims=True)

    q_blk = q_ref[0]
    pT = pT_ref[0]

    qm2 = q_blk * (-2.0)
    pp = (pT[0:1, :] * pT[0:1, :] + pT[1:2, :] * pT[1:2, :]
          + pT[2:3, :] * pT[2:3, :])
    qq = jnp.sum(q_blk * q_blk, axis=1, keepdims=True)
    d = (qm2[:, 0:1] * pT[0:1, :] + qm2[:, 1:2] * pT[1:2, :]
         + qm2[:, 2:3] * pT[2:3, :] + pp)

    inf = jnp.float32(jnp.inf)
    t1 = jnp.min(d, axis=1, keepdims=True)
    dm1 = jnp.where(d == t1, inf, d)
    t2 = jnp.min(dm1, axis=1, keepdims=True)
    dm2 = jnp.where(dm1 == t2, inf, dm1)
    t3 = jnp.min(dm2, axis=1, keepdims=True)

    w1 = 1.0 / jnp.maximum(t1 + qq, 1e-10)
    w2 = 1.0 / jnp.maximum(t2 + qq, 1e-10)
    w3 = 1.0 / jnp.maximum(t3 + qq, 1e-10)
    wsum = w1 + w2 + w3
    wn1 = w1 / wsum
    wn2 = w2 / wsum
    wn3 = w3 / wsum

    ST = jnp.where(d == t1, wn1,
                   jnp.where(dm1 == t2, wn2,
                             jnp.where(dm1 == t3, wn3, 0.0))
                   ).astype(jnp.bfloat16)
    st_ref[i % 2] = ST


def _bn_l2_kernel(h1_ref, s1_ref, ss1_ref, g1_ref, be1_ref, W2_ref, b2_ref,
                  h2_ref, s2_ref, ss2_ref, *, count):
    b = pl.program_id(0)
    j = pl.program_id(1)

    mean = s1_ref[...] / count
    var = ss1_ref[...] / count - mean * mean
    rstd = jax.lax.rsqrt(var + 1e-5)
    scale = g1_ref[...] * rstd
    shift = be1_ref[...] - mean * scale

    h1f = h1_ref[0].astype(jnp.float32)
    h1 = jnp.maximum(h1f * scale + shift, 0.0)
    h2 = (jax.lax.dot_general(W2_ref[...], h1.astype(jnp.bfloat16),
                              dimension_numbers=(((1,), (0,)), ((), ())),
                              preferred_element_type=jnp.float32)
          + b2_ref[...])
    h2_ref[0] = h2.astype(jnp.bfloat16)

    @pl.when(jnp.logical_and(b == 0, j == 0))
    def _():
        s2_ref[...] = jnp.zeros_like(s2_ref)
        ss2_ref[...] = jnp.zeros_like(ss2_ref)

    s2_ref[...] += jnp.sum(h2, axis=1, keepdims=True)
    ss2_ref[...] += jnp.sum(h2 * h2, axis=1, keepdims=True)


def _bn_out_kernel(h2_ref, s2_ref, ss2_ref, g2_ref, be2_ref, out_ref, *,
                   count):
    mean = s2_ref[...] / count
    var = ss2_ref[...] / count - mean * mean
    rstd = jax.lax.rsqrt(var + 1e-5)
    scale = g2_ref[...] * rstd
    shift = be2_ref[...] - mean * scale
    h2f = h2_ref[0].astype(jnp.float32)
    out_ref[0] = jnp.maximum(h2f * scale + shift, 0.0)


def kernel(p, q, x, y, W1, b1, g1, be1, W2, b2, g2, be2):
    B, N, _ = p.shape
    M = q.shape[1]
    Cx = x.shape[1]
    Cy = y.shape[1]
    C1 = W1.shape[0]
    C2 = W2.shape[0]
    count = float(B * M)
    n_blocks = B * (M // BM)
    n_steps = n_blocks + 1
    jb = M // BM

    pT = jnp.swapaxes(p, 1, 2)
    Wx = W1[:, :Cx].astype(jnp.bfloat16)
    Wy = W1[:, Cx:].astype(jnp.bfloat16)
    W2_bf = W2.astype(jnp.bfloat16)
    col = lambda v: v.reshape(-1, 1)

    def cur(i):
        return jnp.minimum(i, n_blocks - 1)

    def prev(i):
        return jnp.maximum(i - 1, 0)

    h1_pre, s1, ss1 = pl.pallas_call(
        functools.partial(_knn_l1_kernel, n_src=N, n_steps=n_steps),
        grid=(n_steps,),
        in_specs=[
            pl.BlockSpec((1, 3, N), lambda i: (cur(i) // jb, 0, 0)),
            pl.BlockSpec((1, BM, 3), lambda i: (cur(i) // jb, cur(i) % jb, 0)),
            pl.BlockSpec((1, Cx, N), lambda i: (prev(i) // jb, 0, 0)),
            pl.BlockSpec((1, Cy, BM), lambda i: (prev(i) // jb, 0,
                                                 prev(i) % jb)),
            pl.BlockSpec((C1, Cx), lambda i: (0, 0)),
            pl.BlockSpec((C1, Cy), lambda i: (0, 0)),
            pl.BlockSpec((C1, 1), lambda i: (0, 0)),
        ],
        out_specs=[
            pl.BlockSpec((1, C1, BM), lambda i: (prev(i) // jb, 0,
                                                 prev(i) % jb)),
            pl.BlockSpec((C1, 1), lambda i: (0, 0)),
            pl.BlockSpec((C1, 1), lambda i: (0, 0)),
        ],
        out_shape=[
            jax.ShapeDtypeStruct((B, C1, M), jnp.bfloat16),
            jax.ShapeDtypeStruct((C1, 1), jnp.float32),
            jax.ShapeDtypeStruct((C1, 1), jnp.float32),
        ],
        scratch_shapes=[pltpu.VMEM((2, BM, N), jnp.bfloat16)],
    )(pT, q, x, y, Wx, Wy, col(b1))

    grid_bc = (B, M // BMC)
    h2_pre, s2, ss2 = pl.pallas_call(
        functools.partial(_bn_l2_kernel, count=count),
        grid=grid_bc,
        in_specs=[
            pl.BlockSpec((1, C1, BMC), lambda b, j: (b, 0, j)),
            pl.BlockSpec((C1, 1), lambda b, j: (0, 0)),
            pl.BlockSpec((C1, 1), lambda b, j: (0, 0)),
            pl.BlockSpec((C1, 1), lambda b, j: (0, 0)),
            pl.BlockSpec((C1, 1), lambda b, j: (0, 0)),
            pl.BlockSpec((C2, C1), lambda b, j: (0, 0)),
            pl.BlockSpec((C2, 1), lambda b, j: (0, 0)),
        ],
        out_specs=[
            pl.BlockSpec((1, C2, BMC), lambda b, j: (b, 0, j)),
            pl.BlockSpec((C2, 1), lambda b, j: (0, 0)),
            pl.BlockSpec((C2, 1), lambda b, j: (0, 0)),
        ],
        out_shape=[
            jax.ShapeDtypeStruct((B, C2, M), jnp.bfloat16),
            jax.ShapeDtypeStruct((C2, 1), jnp.float32),
            jax.ShapeDtypeStruct((C2, 1), jnp.float32),
        ],
    )(h1_pre, s1, ss1, col(g1), col(be1), W2_bf, col(b2))

    h = pl.pallas_call(
        functools.partial(_bn_out_kernel, count=count),
        grid=grid_bc,
        in_specs=[
            pl.BlockSpec((1, C2, BMC), lambda b, j: (b, 0, j)),
            pl.BlockSpec((C2, 1), lambda b, j: (0, 0)),
            pl.BlockSpec((C2, 1), lambda b, j: (0, 0)),
            pl.BlockSpec((C2, 1), lambda b, j: (0, 0)),
            pl.BlockSpec((C2, 1), lambda b, j: (0, 0)),
        ],
        out_specs=pl.BlockSpec((1, C2, BMC), lambda b, j: (b, 0, j)),
        out_shape=jax.ShapeDtypeStruct((B, C2, M), jnp.float32),
    )(h2_pre, s2, ss2, col(g2), col(be2))

    return (q, h)
